# baseline (speedup 1.0000x reference)
import functools
import jax
import jax.numpy as jnp
from jax import lax
from jax.experimental import pallas as pl
from jax.experimental.pallas import tpu as pltpu
from jax.experimental.pallas import tpu_sc as plsc

_N, _E, _D, _G = 10000, 320000, 128, 64
_NP = 10240
_NC, _NS = 2, 16
_NW = _NC * _NS
_EPW = 10240
_EP = _EPW * _NW
_C = 128
_NCHUNK = _EPW // _C
_RPT = _NP // _NS
_DEGW = 16


_K = 4
_HW = _D // _NC
_NCH2 = _EP // (_NS * _C)


def _agg_pipeline(table, sidx, didx, rows, acc, gsem, ssem):
    for b in range(_K - 1):
        pltpu.async_copy(table.at[sidx.at[b]], rows.at[b], gsem.at[b])

    def body(g, carry):
        j0 = g * _K
        for b in range(_K):
            j = j0 + b
            pltpu.make_async_copy(table.at[sidx.at[j]], rows.at[b],
                                  gsem.at[b]).wait()
            pltpu.async_copy(rows.at[b], acc.at[didx.at[j]], ssem.at[b], add=True)
            jn = j + _K - 1
            bn = (b - 1) % _K

            @pl.when(jn < _NCH2)
            def _():
                @pl.when(jn >= _K)
                def _():
                    pltpu.make_async_copy(rows.at[bn], acc.at[didx.at[jn - _K]],
                                          ssem.at[bn]).wait()
                pltpu.async_copy(table.at[sidx.at[jn]], rows.at[bn],
                                 gsem.at[bn])
        return carry

    lax.fori_loop(0, _NCH2 // _K, body, 0)
    for b in range(_K):
        j = _NCH2 - _K + b
        pltpu.make_async_copy(rows.at[b], acc.at[didx.at[j]], ssem.at[b]).wait()


def _edge_body(h_hbm, src_hbm, dst_hbm, out_hbm, sidx, didx, rows, acc,
               gsem, ssem):
    c = lax.axis_index("c")
    s = lax.axis_index("s")
    pltpu.sync_copy(h_hbm.at[c, pl.ds(s * _RPT, _RPT)], acc.at[pl.ds(s * _RPT, _RPT)])
    pltpu.sync_copy(src_hbm.at[pl.ds(s * _NCH2, _NCH2)], sidx)
    pltpu.sync_copy(dst_hbm.at[pl.ds(s * _NCH2, _NCH2)], didx)
    plsc.subcore_barrier()
    _agg_pipeline(h_hbm.at[c], sidx, didx, rows, acc, gsem, ssem)
    plsc.subcore_barrier()
    pltpu.sync_copy(acc.at[pl.ds(s * _RPT, _RPT)], out_hbm.at[c, pl.ds(s * _RPT, _RPT)])


_DGRP = 16
_SCH = 64


def _rsqrt_nr(x):
    i = plsc.bitcast(x, jnp.int32)
    i = jnp.int32(0x5F3759DF) - (i >> 1)
    y = plsc.bitcast(i, jnp.float32)
    for _ in range(3):
        y = y * (1.5 - 0.5 * x * y * y)
    return y


def _conv1_body(h_hbm, src_hbm, dst_hbm, zeros_hbm, hs_hbm, out_hbm, deg_hbm,
                sidx, didx, rows, ones_v, dinvb, hbuf, acc, acc1d, gsem, ssem,
                dsem):
    c = lax.axis_index("c")
    s = lax.axis_index("s")
    slab = pl.ds(s * _RPT, _RPT)
    pltpu.sync_copy(src_hbm.at[pl.ds(s * _NCH2, _NCH2)], sidx)
    pltpu.sync_copy(dst_hbm.at[pl.ds(s * _NCH2, _NCH2)], didx)
    pltpu.sync_copy(zeros_hbm.at[slab], acc1d.at[slab])
    for i in range(_C // 16):
        ones_v[pl.ds(i * 16, 16)] = jnp.ones((16,), jnp.float32)
    plsc.subcore_barrier()

    def deg_body(g, carry):
        for b in range(_DGRP):
            pltpu.async_copy(ones_v, acc1d.at[didx.at[g * _DGRP + b]], dsem,
                             add=True)
        for b in range(_DGRP):
            pltpu.make_async_copy(ones_v, acc1d.at[didx.at[g * _DGRP + b]],
                                  dsem).wait()
        return carry

    lax.fori_loop(0, _NCH2 // _DGRP, deg_body, 0)
    plsc.subcore_barrier()

    pltpu.sync_copy(acc1d.at[slab], dinvb)
    pltpu.sync_copy(dinvb, deg_hbm.at[c, slab])

    def dinv_body(i, carry):
        sl = pl.ds(i * 16, 16)
        dinvb[sl] = _rsqrt_nr(dinvb[sl] + 1.0)
        return carry

    lax.fori_loop(0, _RPT // 16, dinv_body, 0)

    for t in range(_RPT // _SCH):
        r0 = s * _RPT + t * _SCH
        pltpu.sync_copy(h_hbm.at[c, pl.ds(r0, _SCH)], hbuf)

        def row_body(r, carry):
            dv = plsc.load_gather(dinvb, [jnp.full((16,), t * _SCH + r,
                                                   jnp.int32)])
            for k in range(_HW // 16):
                sl = pl.ds(k * 16, 16)
                hbuf[r, sl] = hbuf[r, sl] * dv
            return carry

        lax.fori_loop(0, _SCH, row_body, 0)
        pltpu.sync_copy(hbuf, hs_hbm.at[c, pl.ds(r0, _SCH)])
        pltpu.sync_copy(hbuf, acc.at[pl.ds(r0, _SCH)])
    plsc.subcore_barrier()

    _agg_pipeline(hs_hbm.at[c], sidx, didx, rows, acc, gsem, ssem)
    plsc.subcore_barrier()
    pltpu.sync_copy(acc.at[slab], out_hbm.at[c, slab])


@functools.cache
def _build_conv1_call():
    return pl.kernel(
        _conv1_body,
        out_type=(
            jax.ShapeDtypeStruct((_NC, _NP, _HW), jnp.float32),
            jax.ShapeDtypeStruct((_NC, _NP, _HW), jnp.float32),
            jax.ShapeDtypeStruct((_NC, _NP), jnp.float32),
        ),
        mesh=plsc.VectorSubcoreMesh(core_axis_name="c", subcore_axis_name="s",
                                    num_cores=_NC, num_subcores=_NS),
        scratch_types=[
            pltpu.VMEM((_NCH2, _C), jnp.int32),
            pltpu.VMEM((_NCH2, _C), jnp.int32),
            pltpu.VMEM((_K, _C, _HW), jnp.float32),
            pltpu.VMEM((_C,), jnp.float32),
            pltpu.VMEM((_RPT,), jnp.float32),
            pltpu.VMEM((_SCH, _HW), jnp.float32),
            pltpu.VMEM_SHARED((_NP, _HW), jnp.float32),
            pltpu.VMEM_SHARED((_NP,), jnp.float32),
            pltpu.SemaphoreType.DMA((_K,)),
            pltpu.SemaphoreType.DMA((_K,)),
            pltpu.SemaphoreType.DMA,
        ],
        compiler_params=pltpu.CompilerParams(use_tc_tiling_on_sc=False,
                                             needs_layout_passes=False),
    )


@functools.cache
def _build_edge_call():
    return pl.kernel(
        _edge_body,
        out_type=jax.ShapeDtypeStruct((_NC, _NP, _HW), jnp.float32),
        mesh=plsc.VectorSubcoreMesh(core_axis_name="c", subcore_axis_name="s",
                                    num_cores=_NC, num_subcores=_NS),
        scratch_types=[
            pltpu.VMEM((_NCH2, _C), jnp.int32),
            pltpu.VMEM((_NCH2, _C), jnp.int32),
            pltpu.VMEM((_K, _C, _HW), jnp.float32),
            pltpu.VMEM_SHARED((_NP, _HW), jnp.float32),
            pltpu.SemaphoreType.DMA((_K,)),
            pltpu.SemaphoreType.DMA((_K,)),
        ],
        compiler_params=pltpu.CompilerParams(use_tc_tiling_on_sc=False),
    )


def _dinv(degp_ref):
    deg = degp_ref[0] + 1.0
    return lax.rsqrt(deg)[:, None]


def _k1_body(x_ref, w_ref, b_ref, o_ref):
    h = jnp.dot(x_ref[...], w_ref[...], preferred_element_type=jnp.float32)
    h = h + b_ref[...]
    o_ref[0] = h[:, :_HW]
    o_ref[1] = h[:, _HW:]


def _k2_body(p_ref, degp_ref, bn_ref, w_ref, b_ref, o_ref):
    dinv = _dinv(degp_ref)
    agg = jnp.concatenate([p_ref[0], p_ref[1]], axis=1) * dinv
    g, be, m, v = bn_ref[0], bn_ref[1], bn_ref[2], bn_ref[3]
    z = (agg - m) * lax.rsqrt(v + 1e-5) * g + be
    z = jnp.maximum(z, 0.0)
    h2 = (jnp.dot(z, w_ref[...], preferred_element_type=jnp.float32)
          + b_ref[...]) * dinv
    o_ref[0] = h2[:, :_HW]
    o_ref[1] = h2[:, _HW:]


def _k3_body(p_ref, degp_ref, bn_ref, batch_ref, wc1_ref, bc1_ref,
             wc2_ref, bc2_ref, o_ref):
    dinv = _dinv(degp_ref)
    agg = jnp.concatenate([p_ref[0], p_ref[1]], axis=1) * dinv
    g, be, m, v = bn_ref[0], bn_ref[1], bn_ref[2], bn_ref[3]
    z = (agg - m) * lax.rsqrt(v + 1e-5) * g + be
    z = jnp.maximum(z, 0.0)
    seg = batch_ref[0]
    onehot = (seg[:, None] == lax.broadcasted_iota(jnp.int32, (1, _G), 1)
              ).astype(jnp.float32)
    sums = lax.dot_general(onehot, z, (((0,), (0,)), ((), ())),
                           preferred_element_type=jnp.float32)
    cnt = jnp.sum(onehot, axis=0)
    pool = sums / jnp.maximum(cnt, 1.0)[:, None]
    r = jnp.dot(pool, wc1_ref[...], preferred_element_type=jnp.float32)
    r = jnp.maximum(r + bc1_ref[...], 0.0)
    o_ref[...] = jnp.dot(r, wc2_ref[...], preferred_element_type=jnp.float32) + bc2_ref[...]


def kernel(x, edge_index, batch, W1, b1, g1, be1, m1, v1, W2, b2, g2, be2,
           m2, v2, Wc1, bc1, Wc2, bc2):
    f32 = jnp.float32
    src, dst = edge_index[0], edge_index[1]
    pad = _EP - _E
    padidx = (_N + (jnp.arange(pad, dtype=jnp.int32) % (_NP - _N))).astype(jnp.int32)
    src_p = jnp.concatenate([src, padidx]).reshape(_NW * _NCHUNK, _C)
    dst_p = jnp.concatenate([dst, padidx]).reshape(_NW * _NCHUNK, _C)
    x_p = jnp.pad(x, ((0, _NP - _N), (0, 0)))
    batch_p = jnp.pad(batch, (0, _NP - _N), constant_values=_G).reshape(1, _NP)
    zeros1 = jnp.zeros((_NP,), f32)

    h1 = pl.pallas_call(
        _k1_body,
        out_shape=jax.ShapeDtypeStruct((_NC, _NP, _HW), f32),
    )(x_p, W1, b1.reshape(1, _D))

    _hs, p1, degp = _build_conv1_call()(h1, src_p, dst_p, zeros1)

    bn1 = jnp.stack([g1, be1, m1, v1]).reshape(4, 1, _D)
    h2 = pl.pallas_call(
        _k2_body,
        out_shape=jax.ShapeDtypeStruct((_NC, _NP, _HW), f32),
    )(p1, degp, bn1, W2, b2.reshape(1, _D))

    p2 = _build_edge_call()(h2, src_p, dst_p)

    bn2 = jnp.stack([g2, be2, m2, v2]).reshape(4, 1, _D)
    out = pl.pallas_call(
        _k3_body,
        out_shape=jax.ShapeDtypeStruct((_G, 16), f32),
    )(p2, degp, bn2, batch_p, Wc1, bc1.reshape(1, _G), Wc2,
      bc2.reshape(1, 16))
    return out


# speedup vs baseline: 1.1063x; 1.1063x over previous
import functools
import jax
import jax.numpy as jnp
from jax import lax
from jax.experimental import pallas as pl
from jax.experimental.pallas import tpu as pltpu
from jax.experimental.pallas import tpu_sc as plsc

_N, _E, _D, _G = 10000, 320000, 128, 64
_NP = 10240
_NC, _NS = 2, 16
_NW = _NC * _NS
_EPW = 10240
_EP = _EPW * _NW
_C = 128
_NCHUNK = _EPW // _C
_RPT = _NP // _NS
_DEGW = 16


_DGRP = 16


def _deg_body(dst_hbm, zeros_hbm, ones_hbm, out_hbm, didx, ones_v, acc, sem):
    c = lax.axis_index("c")
    s = lax.axis_index("s")
    w = s * _NC + c
    pltpu.sync_copy(zeros_hbm.at[pl.ds(s * _RPT, _RPT)], acc.at[pl.ds(s * _RPT, _RPT)])
    pltpu.sync_copy(ones_hbm, ones_v)
    pltpu.sync_copy(dst_hbm.at[pl.ds(w * _NCHUNK, _NCHUNK)], didx)
    plsc.subcore_barrier()

    def body(g, carry):
        for b in range(_DGRP):
            pltpu.async_copy(ones_v, acc.at[didx.at[g * _DGRP + b]], sem, add=True)
        for b in range(_DGRP):
            pltpu.make_async_copy(ones_v, acc.at[didx.at[g * _DGRP + b]], sem).wait()
        return carry

    lax.fori_loop(0, _NCHUNK // _DGRP, body, 0)
    plsc.subcore_barrier()
    pltpu.sync_copy(acc.at[pl.ds(s * _RPT, _RPT)], out_hbm.at[c, pl.ds(s * _RPT, _RPT)])


@functools.cache
def _build_deg_call():
    return pl.kernel(
        _deg_body,
        out_type=jax.ShapeDtypeStruct((_NC, _NP), jnp.float32),
        mesh=plsc.VectorSubcoreMesh(core_axis_name="c", subcore_axis_name="s",
                                    num_cores=_NC, num_subcores=_NS),
        scratch_types=[
            pltpu.VMEM((_NCHUNK, _C), jnp.int32),
            pltpu.VMEM((_C,), jnp.float32),
            pltpu.VMEM_SHARED((_NP,), jnp.float32),
            pltpu.SemaphoreType.DMA,
        ],
    )


_K = 5
_HW = _D // _NC
_NCH2 = _EP // (_NS * _C)


def _agg_pipeline(table, sidx, didx, rows, acc, gsem, ssem):
    for b in range(_K - 1):
        pltpu.async_copy(table.at[sidx.at[b]], rows.at[b], gsem.at[b])

    def body(g, carry):
        j0 = g * _K
        for b in range(_K):
            j = j0 + b
            pltpu.make_async_copy(table.at[sidx.at[j]], rows.at[b],
                                  gsem.at[b]).wait()
            pltpu.async_copy(rows.at[b], acc.at[didx.at[j]], ssem.at[b], add=True)
            jn = j + _K - 1
            bn = (b - 1) % _K

            @pl.when(jn < _NCH2)
            def _():
                @pl.when(jn >= _K)
                def _():
                    pltpu.make_async_copy(rows.at[bn], acc.at[didx.at[jn - _K]],
                                          ssem.at[bn]).wait()
                pltpu.async_copy(table.at[sidx.at[jn]], rows.at[bn],
                                 gsem.at[bn])
        return carry

    lax.fori_loop(0, _NCH2 // _K, body, 0)
    for b in range(_K):
        j = _NCH2 - _K + b
        pltpu.make_async_copy(rows.at[b], acc.at[didx.at[j]], ssem.at[b]).wait()


def _edge_body(h_hbm, src_hbm, dst_hbm, out_hbm, sidx, didx, rows, acc,
               gsem, ssem):
    c = lax.axis_index("c")
    s = lax.axis_index("s")
    pltpu.sync_copy(h_hbm.at[c, pl.ds(s * _RPT, _RPT)], acc.at[pl.ds(s * _RPT, _RPT)])
    pltpu.sync_copy(src_hbm.at[pl.ds(s * _NCH2, _NCH2)], sidx)
    pltpu.sync_copy(dst_hbm.at[pl.ds(s * _NCH2, _NCH2)], didx)
    plsc.subcore_barrier()
    _agg_pipeline(h_hbm.at[c], sidx, didx, rows, acc, gsem, ssem)
    plsc.subcore_barrier()
    pltpu.sync_copy(acc.at[pl.ds(s * _RPT, _RPT)], out_hbm.at[c, pl.ds(s * _RPT, _RPT)])


@functools.cache
def _build_edge_call():
    return pl.kernel(
        _edge_body,
        out_type=jax.ShapeDtypeStruct((_NC, _NP, _HW), jnp.float32),
        mesh=plsc.VectorSubcoreMesh(core_axis_name="c", subcore_axis_name="s",
                                    num_cores=_NC, num_subcores=_NS),
        scratch_types=[
            pltpu.VMEM((_NCH2, _C), jnp.int32),
            pltpu.VMEM((_NCH2, _C), jnp.int32),
            pltpu.VMEM((_K, _C, _HW), jnp.float32),
            pltpu.VMEM_SHARED((_NP, _HW), jnp.float32),
            pltpu.SemaphoreType.DMA((_K,)),
            pltpu.SemaphoreType.DMA((_K,)),
        ],
        compiler_params=pltpu.CompilerParams(use_tc_tiling_on_sc=False),
    )


def _dinv(degp_ref):
    deg = degp_ref[0] + degp_ref[1] + 1.0
    return lax.rsqrt(deg)[:, None]


def _k1_body(x_ref, w_ref, b_ref, degp_ref, o_ref):
    h = jnp.dot(x_ref[...], w_ref[...], preferred_element_type=jnp.float32)
    h = (h + b_ref[...]) * _dinv(degp_ref)
    o_ref[0] = h[:, :_HW]
    o_ref[1] = h[:, _HW:]


def _k2_body(p_ref, degp_ref, bn_ref, w_ref, b_ref, o_ref):
    dinv = _dinv(degp_ref)
    agg = jnp.concatenate([p_ref[0], p_ref[1]], axis=1) * dinv
    g, be, m, v = bn_ref[0], bn_ref[1], bn_ref[2], bn_ref[3]
    z = (agg - m) * lax.rsqrt(v + 1e-5) * g + be
    z = jnp.maximum(z, 0.0)
    h2 = (jnp.dot(z, w_ref[...], preferred_element_type=jnp.float32)
          + b_ref[...]) * dinv
    o_ref[0] = h2[:, :_HW]
    o_ref[1] = h2[:, _HW:]


def _k3_body(p_ref, degp_ref, bn_ref, batch_ref, wc1_ref, bc1_ref,
             wc2_ref, bc2_ref, o_ref):
    dinv = _dinv(degp_ref)
    agg = jnp.concatenate([p_ref[0], p_ref[1]], axis=1) * dinv
    g, be, m, v = bn_ref[0], bn_ref[1], bn_ref[2], bn_ref[3]
    z = (agg - m) * lax.rsqrt(v + 1e-5) * g + be
    z = jnp.maximum(z, 0.0)
    seg = batch_ref[0]
    onehot = (seg[:, None] == lax.broadcasted_iota(jnp.int32, (1, _G), 1)
              ).astype(jnp.float32)
    sums = lax.dot_general(onehot, z, (((0,), (0,)), ((), ())),
                           preferred_element_type=jnp.float32)
    cnt = jnp.sum(onehot, axis=0)
    pool = sums / jnp.maximum(cnt, 1.0)[:, None]
    r = jnp.dot(pool, wc1_ref[...], preferred_element_type=jnp.float32)
    r = jnp.maximum(r + bc1_ref[...], 0.0)
    o_ref[...] = jnp.dot(r, wc2_ref[...], preferred_element_type=jnp.float32) + bc2_ref[...]


def kernel(x, edge_index, batch, W1, b1, g1, be1, m1, v1, W2, b2, g2, be2,
           m2, v2, Wc1, bc1, Wc2, bc2):
    f32 = jnp.float32
    src, dst = edge_index[0], edge_index[1]
    pad = _EP - _E
    padidx = (_N + (jnp.arange(pad, dtype=jnp.int32) % (_NP - _N))).astype(jnp.int32)
    src_p = jnp.concatenate([src, padidx]).reshape(_NW * _NCHUNK, _C)
    dst_p = jnp.concatenate([dst, padidx]).reshape(_NW * _NCHUNK, _C)
    x_p = jnp.pad(x, ((0, _NP - _N), (0, 0)))
    batch_p = jnp.pad(batch, (0, _NP - _N), constant_values=_G).reshape(1, _NP)
    zeros1 = jnp.zeros((_NP,), f32)
    ones1 = jnp.ones((_C,), f32)

    degp = _build_deg_call()(dst_p, zeros1, ones1)

    h1 = pl.pallas_call(
        _k1_body,
        out_shape=jax.ShapeDtypeStruct((_NC, _NP, _HW), f32),
    )(x_p, W1, b1.reshape(1, _D), degp)

    p1 = _build_edge_call()(h1, src_p, dst_p)

    bn1 = jnp.stack([g1, be1, m1, v1]).reshape(4, 1, _D)
    h2 = pl.pallas_call(
        _k2_body,
        out_shape=jax.ShapeDtypeStruct((_NC, _NP, _HW), f32),
    )(p1, degp, bn1, W2, b2.reshape(1, _D))

    p2 = _build_edge_call()(h2, src_p, dst_p)

    bn2 = jnp.stack([g2, be2, m2, v2]).reshape(4, 1, _D)
    out = pl.pallas_call(
        _k3_body,
        out_shape=jax.ShapeDtypeStruct((_G, 16), f32),
    )(p2, degp, bn2, batch_p, Wc1, bc1.reshape(1, _G), Wc2,
      bc2.reshape(1, 16))
    return out
